# Initial kernel scaffold; baseline (speedup 1.0000x reference)
import jax
import jax.numpy as jnp
from jax.experimental import pallas as pl


def kernel(x, dw1, pw1, g1, b1, dw2, pw2, g2, b2):
    raise NotImplementedError("write your pallas kernel here")


# speedup vs baseline: 2.6161x; 2.6161x over previous
import functools

import jax
import jax.numpy as jnp
from jax import lax
from jax.experimental import pallas as pl
from jax.experimental.pallas import tpu as pltpu


def _round_up(a, b):
    return -(-a // b) * b


def _s1_kernel(x_ref, i_ref, dw_ref, pw_ref, z1_ref, st_ref, *,
               K, pad, W1, W1r, G):
    TBG, C_in, W = x_ref.shape
    TB = TBG // G
    GCi = G * C_in
    xg = x_ref[...].reshape(TB, GCi, W)
    rows = [lax.dot_general(i_ref[...], xg[j], (((1,), (1,)), ((), ())),
                            preferred_element_type=jnp.float32)
            for j in range(TB)]
    a = jnp.stack(rows, axis=0)
    Wx = max(W + 2 * pad, K + (W1r - 1))
    parts = []
    if pad > 0:
        parts.append(jnp.zeros((TB, pad, GCi), a.dtype))
    parts.append(a)
    if Wx - pad - W > 0:
        parts.append(jnp.zeros((TB, Wx - pad - W, GCi), a.dtype))
    a = jnp.concatenate(parts, axis=1) if len(parts) > 1 else parts[0]
    acc = a[:, 0:W1r, :] * dw_ref[0:1, :][None]
    for k in range(1, K):
        acc = acc + a[:, k:k + W1r, :] * dw_ref[k:k + 1, :][None]
    z = jnp.dot(acc.astype(jnp.bfloat16).reshape(TB * W1r, GCi), pw_ref[...],
                preferred_element_type=jnp.float32)
    z3 = z.reshape(TB, W1r, GCi)
    if W1r != W1:
        col = lax.broadcasted_iota(jnp.int32, (1, W1r, GCi), 1)
        z3 = jnp.where(col < W1, z3, 0.0)
    zb = z3.astype(z1_ref.dtype)
    z1_ref[...] = zb
    zf = zb.astype(jnp.float32).reshape(TB * W1r, GCi)
    s = jnp.sum(zf, axis=0, keepdims=True)
    q = jnp.sum(zf * zf, axis=0, keepdims=True)
    st_ref[...] = jnp.concatenate([s, q], axis=0)[None]


def _s2_kernel(z1_ref, mask_ref, sc_ref, sh_ref, dw_ref, pw_ref,
               z2_ref, st_ref, *, K, pad, W1r, W2, W2r):
    TB, _, GCi = z1_ref.shape
    GCo = z2_ref.shape[2]
    h = jnp.maximum(z1_ref[...] * sc_ref[...][None] + sh_ref[...][None], 0.0)
    h = h * mask_ref[...]
    Whp = max(2 * pad + W1r, W2r + K - 1)
    parts = []
    if pad > 0:
        parts.append(jnp.zeros((TB, pad, GCi), h.dtype))
    parts.append(h)
    rpad = Whp - pad - W1r
    if rpad > 0:
        parts.append(jnp.zeros((TB, rpad, GCi), h.dtype))
    hp = jnp.concatenate(parts, axis=1) if len(parts) > 1 else parts[0]
    acc = hp[:, 0:W2r, :] * dw_ref[0:1, :][None]
    for k in range(1, K):
        acc = acc + hp[:, k:k + W2r, :] * dw_ref[k:k + 1, :][None]
    z = jnp.dot(acc.astype(jnp.bfloat16).reshape(TB * W2r, GCi), pw_ref[...],
                preferred_element_type=jnp.float32)
    z3 = z.reshape(TB, W2r, GCo)
    if W2r != W2:
        col = lax.broadcasted_iota(jnp.int32, (1, W2r, GCo), 1)
        z3 = jnp.where(col < W2, z3, 0.0)
    zb = z3.astype(z2_ref.dtype)
    z2_ref[...] = zb
    zf = zb.astype(jnp.float32).reshape(TB * W2r, GCo)
    s = jnp.sum(zf, axis=0, keepdims=True)
    q = jnp.sum(zf * zf, axis=0, keepdims=True)
    st_ref[...] = jnp.concatenate([s, q], axis=0)[None]


def _s3_kernel(z2_ref, i_ref, sc_ref, sh_ref, o_ref, *, NPB, RPB, G, C_out, W2r):
    sc = jnp.transpose(sc_ref[...])
    sh = jnp.transpose(sh_ref[...])
    for nn in range(NPB):
        for r in range(RPB):
            zt = lax.dot_general(z2_ref[nn * RPB + r], i_ref[...],
                                 (((0,), (0,)), ((), ())),
                                 preferred_element_type=jnp.float32)
            zt = zt * sc + sh
            for g in range(G):
                o_ref[nn, :, r * G + g, :] = zt[g * C_out:(g + 1) * C_out, :]


def kernel(x, dw1, pw1, g1, b1, dw2, pw2, g2, b2, *,
           kernel_size=3, stride=1, padding=1, eps=1e-5):
    f32 = jnp.float32
    bf16 = jnp.bfloat16
    N, C_in, H, W = x.shape
    C_out = pw2.shape[0]
    K, pad = kernel_size, padding

    xh = x[:, :, ::stride, :]
    Hh = xh.shape[2]
    B = N * Hh
    W1 = (W + 2 * pad - K) // stride + 1
    W2 = W1 + 2 * pad - K + 1
    W1r = _round_up(W1, 8)
    W2r = _round_up(W2, 8)
    Wx = max(W + 2 * pad, K + stride * (W1r - 1))
    assert stride == 1, "specialized for stride 1"

    G = max(1, 128 // C_in)
    GCi, GCo = G * C_in, G * C_out
    Bg0 = -(-B // G)
    TBg = min(64, Bg0)
    n_tiles = -(-Bg0 // TBg)
    Bg = n_tiles * TBg
    Bp = Bg * G

    xt = jnp.transpose(xh, (0, 2, 1, 3)).reshape(B, C_in, W)
    xt = jnp.pad(jnp.maximum(xt, 0.0), ((0, Bp - B), (0, 0), (0, 0)))
    xt = xt.astype(bf16)

    dw1_l = jnp.tile(jnp.transpose(dw1[:, 0, 0, :]), (1, G)).astype(f32)
    dw2_l = jnp.tile(jnp.transpose(dw2[:, 0, 0, :]), (1, G)).astype(f32)
    pw1_bd = jnp.kron(jnp.eye(G, dtype=f32),
                      jnp.transpose(pw1[:, :, 0, 0])).astype(bf16)
    pw2_bd = jnp.kron(jnp.eye(G, dtype=f32),
                      jnp.transpose(pw2[:, :, 0, 0])).astype(bf16)

    r_idx = jnp.arange(Bg)[:, None]
    g_idx = jnp.repeat(jnp.arange(G), C_in)[None, :]
    row_mask = ((r_idx * G + g_idx) < B).astype(f32)[:, None, :]

    vmem_limit = 48 * 1024 * 1024
    cparams = pltpu.CompilerParams(dimension_semantics=("parallel",),
                                   vmem_limit_bytes=vmem_limit)

    eye_in = jnp.eye(W, dtype=bf16)
    z1p, st1 = pl.pallas_call(
        functools.partial(_s1_kernel, K=K, pad=pad, W1=W1, W1r=W1r, G=G),
        grid=(n_tiles,),
        in_specs=[pl.BlockSpec((TBg * G, C_in, W), lambda i: (i, 0, 0)),
                  pl.BlockSpec((W, W), lambda i: (0, 0)),
                  pl.BlockSpec((K, GCi), lambda i: (0, 0)),
                  pl.BlockSpec((GCi, GCi), lambda i: (0, 0))],
        out_specs=[pl.BlockSpec((TBg, W1r, GCi), lambda i: (i, 0, 0)),
                   pl.BlockSpec((1, 2, GCi), lambda i: (i, 0, 0))],
        out_shape=[jax.ShapeDtypeStruct((Bg, W1r, GCi), bf16),
                   jax.ShapeDtypeStruct((n_tiles, 2, GCi), f32)],
        compiler_params=cparams,
    )(xt, eye_in, dw1_l, pw1_bd)

    cnt1 = float(B * W1)
    cnt2 = float(B * W2)

    s1 = st1[:, 0, :].sum(0).reshape(G, C_in).sum(0)
    q1 = st1[:, 1, :].sum(0).reshape(G, C_in).sum(0)
    m1 = s1 / cnt1
    v1 = jnp.maximum(q1 / cnt1 - m1 * m1, 0.0)
    sc1 = g1 * lax.rsqrt(v1 + eps)
    sh1 = b1 - m1 * sc1
    sc1_l = jnp.tile(sc1, G).reshape(1, GCi).astype(f32)
    sh1_l = jnp.tile(sh1, G).reshape(1, GCi).astype(f32)

    z2p, st2 = pl.pallas_call(
        functools.partial(_s2_kernel, K=K, pad=pad, W1r=W1r, W2=W2, W2r=W2r),
        grid=(n_tiles,),
        in_specs=[pl.BlockSpec((TBg, W1r, GCi), lambda i: (i, 0, 0)),
                  pl.BlockSpec((TBg, 1, GCi), lambda i: (i, 0, 0)),
                  pl.BlockSpec((1, GCi), lambda i: (0, 0)),
                  pl.BlockSpec((1, GCi), lambda i: (0, 0)),
                  pl.BlockSpec((K, GCi), lambda i: (0, 0)),
                  pl.BlockSpec((GCi, GCo), lambda i: (0, 0))],
        out_specs=[pl.BlockSpec((TBg, W2r, GCo), lambda i: (i, 0, 0)),
                   pl.BlockSpec((1, 2, GCo), lambda i: (i, 0, 0))],
        out_shape=[jax.ShapeDtypeStruct((Bg, W2r, GCo), bf16),
                   jax.ShapeDtypeStruct((n_tiles, 2, GCo), f32)],
        compiler_params=cparams,
    )(z1p, row_mask, sc1_l, sh1_l, dw2_l, pw2_bd)

    s2 = st2[:, 0, :].sum(0).reshape(G, C_out).sum(0)
    q2 = st2[:, 1, :].sum(0).reshape(G, C_out).sum(0)
    m2 = s2 / cnt2
    v2 = jnp.maximum(q2 / cnt2 - m2 * m2, 0.0)
    sc2 = g2 * lax.rsqrt(v2 + eps)
    sh2 = b2 - m2 * sc2

    if Hh % G == 0 and Bg == Bg0 and W2r == W2:
        RPB = Hh // G
        NPB = 4 if N % 4 == 0 else (2 if N % 2 == 0 else 1)
        sc2_l = jnp.tile(sc2, G).reshape(1, GCo).astype(f32)
        sh2_l = jnp.tile(sh2, G).reshape(1, GCo).astype(f32)
        eye_w = jnp.eye(W2r, dtype=bf16)
        out = pl.pallas_call(
            functools.partial(_s3_kernel, NPB=NPB, RPB=RPB, G=G,
                              C_out=C_out, W2r=W2r),
            grid=(N // NPB,),
            in_specs=[pl.BlockSpec((NPB * RPB, W2r, GCo), lambda i: (i, 0, 0)),
                      pl.BlockSpec((W2r, W2r), lambda i: (0, 0)),
                      pl.BlockSpec((1, GCo), lambda i: (0, 0)),
                      pl.BlockSpec((1, GCo), lambda i: (0, 0))],
            out_specs=pl.BlockSpec((NPB, C_out, Hh, W2), lambda i: (i, 0, 0, 0)),
            out_shape=jax.ShapeDtypeStruct((N, C_out, Hh, W2), f32),
            compiler_params=cparams,
        )(z2p, eye_w, sc2_l, sh2_l)
        return out.astype(x.dtype)

    y = z2p.reshape(Bg, W2r, G, C_out).transpose(0, 2, 1, 3).reshape(Bp, W2r, C_out)
    y = y[:B, :W2, :].astype(f32) * sc2[None, None, :] + sh2[None, None, :]
    out = y.reshape(N, Hh, W2, C_out).transpose(0, 3, 1, 2)
    return out.astype(x.dtype)


# speedup vs baseline: 2.6444x; 1.0108x over previous
import functools

import jax
import jax.numpy as jnp
from jax import lax
from jax.experimental import pallas as pl
from jax.experimental.pallas import tpu as pltpu


def _round_up(a, b):
    return -(-a // b) * b


def _s1_kernel(x_ref, i_ref, dw_ref, pw_ref, z1_ref, st_ref, *,
               K, pad, W1, W1r, G):
    TBG, C_in, W = x_ref.shape
    TB = TBG // G
    GCi = G * C_in
    xg = x_ref[...].reshape(TB, GCi, W)
    rows = [lax.dot_general(i_ref[...], xg[j], (((1,), (1,)), ((), ())),
                            preferred_element_type=jnp.float32)
            for j in range(TB)]
    a = jnp.stack(rows, axis=0)
    Wx = max(W + 2 * pad, K + (W1r - 1))
    parts = []
    if pad > 0:
        parts.append(jnp.zeros((TB, pad, GCi), a.dtype))
    parts.append(a)
    if Wx - pad - W > 0:
        parts.append(jnp.zeros((TB, Wx - pad - W, GCi), a.dtype))
    a = jnp.concatenate(parts, axis=1) if len(parts) > 1 else parts[0]
    acc = a[:, 0:W1r, :] * dw_ref[0:1, :][None]
    for k in range(1, K):
        acc = acc + a[:, k:k + W1r, :] * dw_ref[k:k + 1, :][None]
    z = jnp.dot(acc.astype(jnp.bfloat16).reshape(TB * W1r, GCi), pw_ref[...],
                preferred_element_type=jnp.float32)
    z3 = z.reshape(TB, W1r, GCi)
    if W1r != W1:
        col = lax.broadcasted_iota(jnp.int32, (1, W1r, GCi), 1)
        z3 = jnp.where(col < W1, z3, 0.0)
    zb = z3.astype(z1_ref.dtype)
    z1_ref[...] = zb
    zf = zb.astype(jnp.float32).reshape(TB * W1r, GCi)
    s = jnp.sum(zf, axis=0, keepdims=True)
    q = jnp.sum(zf * zf, axis=0, keepdims=True)
    st_ref[...] = jnp.concatenate([s, q], axis=0)[None]


def _s2_kernel(z1_ref, mask_ref, sc_ref, sh_ref, dw_ref, pw_ref,
               z2_ref, st_ref, *, K, pad, W1r, W2, W2r, use_mask):
    TB, _, GCi = z1_ref.shape
    GCo = z2_ref.shape[2]
    h = jnp.maximum(z1_ref[...] * sc_ref[...][None] + sh_ref[...][None], 0.0)
    if use_mask:
        h = h * mask_ref[...]
    Whp = max(2 * pad + W1r, W2r + K - 1)
    parts = []
    if pad > 0:
        parts.append(jnp.zeros((TB, pad, GCi), h.dtype))
    parts.append(h)
    rpad = Whp - pad - W1r
    if rpad > 0:
        parts.append(jnp.zeros((TB, rpad, GCi), h.dtype))
    hp = jnp.concatenate(parts, axis=1) if len(parts) > 1 else parts[0]
    acc = hp[:, 0:W2r, :] * dw_ref[0:1, :][None]
    for k in range(1, K):
        acc = acc + hp[:, k:k + W2r, :] * dw_ref[k:k + 1, :][None]
    z = jnp.dot(acc.astype(jnp.bfloat16).reshape(TB * W2r, GCi), pw_ref[...],
                preferred_element_type=jnp.float32)
    z3 = z.reshape(TB, W2r, GCo)
    if W2r != W2:
        col = lax.broadcasted_iota(jnp.int32, (1, W2r, GCo), 1)
        z3 = jnp.where(col < W2, z3, 0.0)
    zb = z3.astype(z2_ref.dtype)
    z2_ref[...] = zb
    zf = zb.astype(jnp.float32).reshape(TB * W2r, GCo)
    s = jnp.sum(zf, axis=0, keepdims=True)
    q = jnp.sum(zf * zf, axis=0, keepdims=True)
    st_ref[...] = jnp.concatenate([s, q], axis=0)[None]


def _s3_kernel(z2_ref, i_ref, sc_ref, sh_ref, o_ref, *, NPB, RPB, G, C_out, W2r):
    sc = jnp.transpose(sc_ref[...])
    sh = jnp.transpose(sh_ref[...])
    for nn in range(NPB):
        for r in range(RPB):
            zt = lax.dot_general(z2_ref[nn * RPB + r], i_ref[...],
                                 (((0,), (0,)), ((), ())),
                                 preferred_element_type=jnp.float32)
            zt = zt * sc + sh
            for g in range(G):
                o_ref[nn, :, r * G + g, :] = zt[g * C_out:(g + 1) * C_out, :]


def kernel(x, dw1, pw1, g1, b1, dw2, pw2, g2, b2, *,
           kernel_size=3, stride=1, padding=1, eps=1e-5):
    f32 = jnp.float32
    bf16 = jnp.bfloat16
    N, C_in, H, W = x.shape
    C_out = pw2.shape[0]
    K, pad = kernel_size, padding

    xh = x[:, :, ::stride, :]
    Hh = xh.shape[2]
    B = N * Hh
    W1 = (W + 2 * pad - K) // stride + 1
    W2 = W1 + 2 * pad - K + 1
    W1r = _round_up(W1, 8)
    W2r = _round_up(W2, 8)
    Wx = max(W + 2 * pad, K + stride * (W1r - 1))
    assert stride == 1, "specialized for stride 1"

    G = max(1, 128 // C_in)
    GCi, GCo = G * C_in, G * C_out
    Bg0 = -(-B // G)
    TBg = min(64, Bg0)
    n_tiles = -(-Bg0 // TBg)
    Bg = n_tiles * TBg
    Bp = Bg * G

    xt = jnp.transpose(xh, (0, 2, 1, 3)).reshape(B, C_in, W)
    xt = jnp.pad(jnp.maximum(xt, 0.0), ((0, Bp - B), (0, 0), (0, 0)))
    xt = xt.astype(bf16)

    dw1_l = jnp.tile(jnp.transpose(dw1[:, 0, 0, :]), (1, G)).astype(f32)
    dw2_l = jnp.tile(jnp.transpose(dw2[:, 0, 0, :]), (1, G)).astype(f32)
    pw1_bd = jnp.kron(jnp.eye(G, dtype=f32),
                      jnp.transpose(pw1[:, :, 0, 0])).astype(bf16)
    pw2_bd = jnp.kron(jnp.eye(G, dtype=f32),
                      jnp.transpose(pw2[:, :, 0, 0])).astype(bf16)

    r_idx = jnp.arange(Bg)[:, None]
    g_idx = jnp.repeat(jnp.arange(G), C_in)[None, :]
    row_mask = ((r_idx * G + g_idx) < B).astype(f32)[:, None, :]

    vmem_limit = 48 * 1024 * 1024
    cparams = pltpu.CompilerParams(dimension_semantics=("parallel",),
                                   vmem_limit_bytes=vmem_limit)

    eye_in = jnp.eye(W, dtype=bf16)
    z1p, st1 = pl.pallas_call(
        functools.partial(_s1_kernel, K=K, pad=pad, W1=W1, W1r=W1r, G=G),
        grid=(n_tiles,),
        in_specs=[pl.BlockSpec((TBg * G, C_in, W), lambda i: (i, 0, 0)),
                  pl.BlockSpec((W, W), lambda i: (0, 0)),
                  pl.BlockSpec((K, GCi), lambda i: (0, 0)),
                  pl.BlockSpec((GCi, GCi), lambda i: (0, 0))],
        out_specs=[pl.BlockSpec((TBg, W1r, GCi), lambda i: (i, 0, 0)),
                   pl.BlockSpec((1, 2, GCi), lambda i: (i, 0, 0))],
        out_shape=[jax.ShapeDtypeStruct((Bg, W1r, GCi), bf16),
                   jax.ShapeDtypeStruct((n_tiles, 2, GCi), f32)],
        compiler_params=cparams,
    )(xt, eye_in, dw1_l, pw1_bd)

    cnt1 = float(B * W1)
    cnt2 = float(B * W2)

    s1 = st1[:, 0, :].sum(0).reshape(G, C_in).sum(0)
    q1 = st1[:, 1, :].sum(0).reshape(G, C_in).sum(0)
    m1 = s1 / cnt1
    v1 = jnp.maximum(q1 / cnt1 - m1 * m1, 0.0)
    sc1 = g1 * lax.rsqrt(v1 + eps)
    sh1 = b1 - m1 * sc1
    sc1_l = jnp.tile(sc1, G).reshape(1, GCi).astype(f32)
    sh1_l = jnp.tile(sh1, G).reshape(1, GCi).astype(f32)

    z2p, st2 = pl.pallas_call(
        functools.partial(_s2_kernel, K=K, pad=pad, W1r=W1r, W2=W2, W2r=W2r,
                          use_mask=Bp > B),
        grid=(n_tiles,),
        in_specs=[pl.BlockSpec((TBg, W1r, GCi), lambda i: (i, 0, 0)),
                  pl.BlockSpec((TBg, 1, GCi), lambda i: (i, 0, 0)),
                  pl.BlockSpec((1, GCi), lambda i: (0, 0)),
                  pl.BlockSpec((1, GCi), lambda i: (0, 0)),
                  pl.BlockSpec((K, GCi), lambda i: (0, 0)),
                  pl.BlockSpec((GCi, GCo), lambda i: (0, 0))],
        out_specs=[pl.BlockSpec((TBg, W2r, GCo), lambda i: (i, 0, 0)),
                   pl.BlockSpec((1, 2, GCo), lambda i: (i, 0, 0))],
        out_shape=[jax.ShapeDtypeStruct((Bg, W2r, GCo), bf16),
                   jax.ShapeDtypeStruct((n_tiles, 2, GCo), f32)],
        compiler_params=cparams,
    )(z1p, row_mask, sc1_l, sh1_l, dw2_l, pw2_bd)

    s2 = st2[:, 0, :].sum(0).reshape(G, C_out).sum(0)
    q2 = st2[:, 1, :].sum(0).reshape(G, C_out).sum(0)
    m2 = s2 / cnt2
    v2 = jnp.maximum(q2 / cnt2 - m2 * m2, 0.0)
    sc2 = g2 * lax.rsqrt(v2 + eps)
    sh2 = b2 - m2 * sc2

    if Hh % G == 0 and Bg == Bg0 and W2r == W2:
        RPB = Hh // G
        NPB = 4 if N % 4 == 0 else (2 if N % 2 == 0 else 1)
        sc2_l = jnp.tile(sc2, G).reshape(1, GCo).astype(f32)
        sh2_l = jnp.tile(sh2, G).reshape(1, GCo).astype(f32)
        eye_w = jnp.eye(W2r, dtype=bf16)
        out = pl.pallas_call(
            functools.partial(_s3_kernel, NPB=NPB, RPB=RPB, G=G,
                              C_out=C_out, W2r=W2r),
            grid=(N // NPB,),
            in_specs=[pl.BlockSpec((NPB * RPB, W2r, GCo), lambda i: (i, 0, 0)),
                      pl.BlockSpec((W2r, W2r), lambda i: (0, 0)),
                      pl.BlockSpec((1, GCo), lambda i: (0, 0)),
                      pl.BlockSpec((1, GCo), lambda i: (0, 0))],
            out_specs=pl.BlockSpec((NPB, C_out, Hh, W2), lambda i: (i, 0, 0, 0)),
            out_shape=jax.ShapeDtypeStruct((N, C_out, Hh, W2), f32),
            compiler_params=cparams,
        )(z2p, eye_w, sc2_l, sh2_l)
        return out.astype(x.dtype)

    y = z2p.reshape(Bg, W2r, G, C_out).transpose(0, 2, 1, 3).reshape(Bp, W2r, C_out)
    y = y[:B, :W2, :].astype(f32) * sc2[None, None, :] + sh2[None, None, :]
    out = y.reshape(N, Hh, W2, C_out).transpose(0, 3, 1, 2)
    return out.astype(x.dtype)
